# Initial kernel scaffold; baseline (speedup 1.0000x reference)
import jax
import jax.numpy as jnp
from jax.experimental import pallas as pl


def kernel(x, edge_index, edge_attr, W1, b1, W2, b2, W3, b3):
    raise NotImplementedError("write your pallas kernel here")


# speedup vs baseline: 22.0019x; 22.0019x over previous
import functools

import jax
import jax.numpy as jnp
from jax import lax
from jax.experimental import pallas as pl
from jax.experimental.pallas import tpu as pltpu
from jax.experimental.pallas import tpu_sc as plsc

N = 10000
E = 320000
D = 128

NC = 2
NS = 16
LANES = 16
NW = NC * NS
K = 80
CHUNKS = E // (NW * K)
ROWS_PER_W = E // (NW * K)
NPAD = 10240
NSLC = NPAD // NS

_MESH = plsc.VectorSubcoreMesh(core_axis_name="c", subcore_axis_name="s")


def _deg_body(col_hbm, ew_hbm, zero_hbm, out_hbm, colv, ewv, acc):
    cid = lax.axis_index("c")
    sid = lax.axis_index("s")
    wid = sid * NC + cid
    pltpu.sync_copy(zero_hbm, acc.at[pl.ds(sid * NSLC, NSLC)])
    plsc.subcore_barrier()
    pltpu.sync_copy(col_hbm.at[wid], colv)
    pltpu.sync_copy(ew_hbm.at[wid], ewv)

    def chunk(j, carry):
        pltpu.sync_copy(ewv.at[j], acc.at[colv.at[j]], add=True)
        return carry

    lax.fori_loop(0, CHUNKS, chunk, 0)
    plsc.subcore_barrier()
    pltpu.sync_copy(acc.at[pl.ds(sid * NSLC, NSLC)],
                    out_hbm.at[cid, pl.ds(sid * NSLC, NSLC)])


@functools.partial(
    pl.kernel,
    mesh=_MESH,
    out_type=jax.ShapeDtypeStruct((NC, NPAD), jnp.float32),
    scratch_types=[
        pltpu.VMEM((ROWS_PER_W, K), jnp.int32),
        pltpu.VMEM((ROWS_PER_W, K), jnp.float32),
        pltpu.VMEM_SHARED((NPAD,), jnp.float32),
    ],
    compiler_params=pltpu.CompilerParams(needs_layout_passes=False, use_tc_tiling_on_sc=False),
)
def _deg_kernel(col_hbm, ew_hbm, zero_hbm, out_hbm, colv, ewv, acc):
    _deg_body(col_hbm, ew_hbm, zero_hbm, out_hbm, colv, ewv, acc)


def _make_layer_kernel(H):
    groups = (K * H) // LANES

    def body(y_hbm, row_hbm, col_hbm, ew_hbm, zero_hbm, out_hbm,
             rowv, colv, ewv, gbuf, acc, sem):
        cid = lax.axis_index("c")
        sid = lax.axis_index("s")
        wid = sid * NC + cid
        pltpu.sync_copy(zero_hbm, acc.at[pl.ds(sid * NSLC, NSLC)])
        plsc.subcore_barrier()
        pltpu.sync_copy(row_hbm.at[wid], rowv)
        pltpu.sync_copy(col_hbm.at[wid], colv)
        pltpu.sync_copy(ew_hbm.at[wid], ewv)

        lanes = lax.iota(jnp.int32, LANES)
        log2h = H.bit_length() - 1
        e_base = lax.shift_right_logical(lanes, log2h)
        h_vec = lax.bitwise_and(lanes, H - 1)

        def chunk(j, carry):
            pltpu.async_copy(y_hbm.at[rowv.at[j]], gbuf, sem).wait()
            ew_row = ewv.at[j]
            if H == LANES:
                for g in range(groups):
                    if g % LANES == 0:
                        ew16 = ew_row[pl.ds(g, LANES)]
                    gbuf[g, :] = gbuf[g, :] * ew16[g % LANES]
            else:
                for g in range(groups):
                    e_vec = e_base + (LANES * g) // H
                    v = plsc.load_gather(gbuf, [e_vec, h_vec])
                    s = plsc.load_gather(ew_row, [e_vec])
                    plsc.store_scatter(gbuf, [e_vec, h_vec], v * s)
            pltpu.sync_copy(gbuf, acc.at[colv.at[j]], add=True)
            return carry

        lax.fori_loop(0, CHUNKS, chunk, 0)
        plsc.subcore_barrier()
        pltpu.sync_copy(acc.at[pl.ds(sid * NSLC, NSLC)],
                        out_hbm.at[cid, pl.ds(sid * NSLC, NSLC)])

    return pl.kernel(
        body,
        mesh=_MESH,
        out_type=jax.ShapeDtypeStruct((NC, NPAD, H), jnp.float32),
        scratch_types=[
            pltpu.VMEM((ROWS_PER_W, K), jnp.int32),
            pltpu.VMEM((ROWS_PER_W, K), jnp.int32),
            pltpu.VMEM((ROWS_PER_W, K), jnp.float32),
            pltpu.VMEM((K, H), jnp.float32),
            pltpu.VMEM_SHARED((NPAD, H), jnp.float32),
            pltpu.SemaphoreType.DMA,
        ],
        compiler_params=pltpu.CompilerParams(needs_layout_passes=False, use_tc_tiling_on_sc=False),
    )


_layer_8 = _make_layer_kernel(8)
_layer_16 = _make_layer_kernel(16)


def _elu(v):
    return jnp.where(v > 0, v, jnp.exp(v) - 1.0)


def _tc1_body(degp_ref, x_ref, w1_ref, dinv_ref, y1_ref):
    deg = degp_ref[0, :] + degp_ref[1, :] + 1.0
    dinv = lax.rsqrt(deg)
    dinv_ref[...] = dinv[:, None]
    y1_ref[...] = jnp.dot(x_ref[...], w1_ref[...],
                          preferred_element_type=jnp.float32) * dinv[:, None]


def _tc_mid_body(sp_ref, y_ref, dinv_ref, b_ref, w_ref, yn_ref):
    dinv = dinv_ref[...]
    h = _elu(dinv * (sp_ref[0] + sp_ref[1] + y_ref[...]) + b_ref[...][None, :])
    yn_ref[...] = jnp.dot(h, w_ref[...],
                          preferred_element_type=jnp.float32) * dinv


def _tc_final_body(sp_ref, y_ref, dinv_ref, b_ref, out_ref):
    out_ref[...] = (dinv_ref[...] * (sp_ref[0] + sp_ref[1] + y_ref[...])
                    + b_ref[...][None, :])


def kernel(x, edge_index, edge_attr, W1, b1, W2, b2, W3, b3):
    row2d = edge_index[0].reshape(NW, ROWS_PER_W, K)
    col2d = edge_index[1].reshape(NW, ROWS_PER_W, K)
    ew2d = edge_attr.reshape(NW, ROWS_PER_W, K)
    zero1 = jnp.zeros((NSLC,), jnp.float32)

    deg_parts = _deg_kernel(col2d, ew2d, zero1)
    degp = deg_parts[:, :N]

    dinv, y1 = pl.pallas_call(
        _tc1_body,
        out_shape=(jax.ShapeDtypeStruct((N, 1), jnp.float32),
                   jax.ShapeDtypeStruct((N, 8), jnp.float32)),
    )(degp, x, W1)

    s1 = _layer_8(y1, row2d, col2d, ew2d, jnp.zeros((NSLC, 8), jnp.float32))
    W2p = jnp.pad(W2, ((0, 0), (0, 4)))
    y2p = pl.pallas_call(
        _tc_mid_body,
        out_shape=jax.ShapeDtypeStruct((N, 8), jnp.float32),
    )(s1[:, :N, :], y1, dinv, b1, W2p)

    s2 = _layer_8(y2p, row2d, col2d, ew2d, jnp.zeros((NSLC, 8), jnp.float32))
    y3 = pl.pallas_call(
        _tc_mid_body,
        out_shape=jax.ShapeDtypeStruct((N, 16), jnp.float32),
    )(s2[:, :N, :4], y2p[:, :4], dinv, b2, W3)

    s3 = _layer_16(y3, row2d, col2d, ew2d, jnp.zeros((NSLC, 16), jnp.float32))
    out = pl.pallas_call(
        _tc_final_body,
        out_shape=jax.ShapeDtypeStruct((N, 16), jnp.float32),
    )(s3[:, :N, :], y3, dinv, b3)
    return out


# speedup vs baseline: 28.6373x; 1.3016x over previous
import functools

import jax
import jax.numpy as jnp
from jax import lax
from jax.experimental import pallas as pl
from jax.experimental.pallas import tpu as pltpu
from jax.experimental.pallas import tpu_sc as plsc

N = 10000
E = 320000
D = 128

NC = 2
NS = 16
LANES = 16
NW = NC * NS
K = 128
ROWS_PER_W = 80
CHUNKS = ROWS_PER_W
EPAD = NW * ROWS_PER_W * K
NBUF = 5
OUTER = CHUNKS // NBUF
NPAD = 10240
NSLC = NPAD // NS

_MESH = plsc.VectorSubcoreMesh(core_axis_name="c", subcore_axis_name="s")


def _deg_body(col_hbm, ew_hbm, zero_hbm, out_hbm, colv, ewv, acc):
    cid = lax.axis_index("c")
    sid = lax.axis_index("s")
    wid = sid * NC + cid
    pltpu.sync_copy(zero_hbm, acc.at[pl.ds(sid * NSLC, NSLC)])
    plsc.subcore_barrier()
    pltpu.sync_copy(col_hbm.at[wid], colv)
    pltpu.sync_copy(ew_hbm.at[wid], ewv)

    def chunk(j, carry):
        pltpu.sync_copy(ewv.at[j], acc.at[colv.at[j]], add=True)
        return carry

    lax.fori_loop(0, CHUNKS, chunk, 0)
    plsc.subcore_barrier()
    pltpu.sync_copy(acc.at[pl.ds(sid * NSLC, NSLC)],
                    out_hbm.at[cid, pl.ds(sid * NSLC, NSLC)])


@functools.partial(
    pl.kernel,
    mesh=_MESH,
    out_type=jax.ShapeDtypeStruct((NC, NPAD), jnp.float32),
    scratch_types=[
        pltpu.VMEM((ROWS_PER_W, K), jnp.int32),
        pltpu.VMEM((ROWS_PER_W, K), jnp.float32),
        pltpu.VMEM_SHARED((NPAD,), jnp.float32),
    ],
    compiler_params=pltpu.CompilerParams(needs_layout_passes=False, use_tc_tiling_on_sc=False),
)
def _deg_kernel(col_hbm, ew_hbm, zero_hbm, out_hbm, colv, ewv, acc):
    _deg_body(col_hbm, ew_hbm, zero_hbm, out_hbm, colv, ewv, acc)


def _make_layer_kernel(H):
    groups = (K * H) // LANES

    def body(y_hbm, row_hbm, col_hbm, ew_hbm, zero_hbm, out_hbm,
             rowv, colv, ewv, gbuf, acc, semg, sems):
        cid = lax.axis_index("c")
        sid = lax.axis_index("s")
        wid = sid * NC + cid
        pltpu.sync_copy(zero_hbm, acc.at[pl.ds(sid * NSLC, NSLC)])
        plsc.subcore_barrier()
        pltpu.sync_copy(row_hbm.at[wid], rowv)
        pltpu.sync_copy(col_hbm.at[wid], colv)
        pltpu.sync_copy(ew_hbm.at[wid], ewv)

        lanes = lax.iota(jnp.int32, LANES)
        log2h = H.bit_length() - 1
        e_base = lax.shift_right_logical(lanes, log2h)
        h_vec = lax.bitwise_and(lanes, H - 1)

        def chunk_group(i, carry):
            base = i * NBUF
            gds = [pltpu.async_copy(y_hbm.at[rowv.at[base + b]], gbuf.at[b],
                                    semg.at[b]) for b in range(NBUF)]
            sds = []
            for b in range(NBUF):
                gds[b].wait()
                gb = gbuf.at[b]
                ew_row = ewv.at[base + b]
                if H == LANES:
                    for g in range(groups):
                        if g % LANES == 0:
                            ew16 = ew_row[pl.ds(g, LANES)]
                        gb[g, :] = gb[g, :] * ew16[g % LANES]
                else:
                    for g in range(groups):
                        e_vec = e_base + (LANES * g) // H
                        v = plsc.load_gather(gb, [e_vec, h_vec])
                        s = plsc.load_gather(ew_row, [e_vec])
                        plsc.store_scatter(gb, [e_vec, h_vec], v * s)
                sds.append(pltpu.async_copy(gb, acc.at[colv.at[base + b]],
                                            sems.at[b], add=True))
            for sd in sds:
                sd.wait()
            return carry

        lax.fori_loop(0, OUTER, chunk_group, 0)
        plsc.subcore_barrier()
        pltpu.sync_copy(acc.at[pl.ds(sid * NSLC, NSLC)],
                        out_hbm.at[cid, pl.ds(sid * NSLC, NSLC)])

    return pl.kernel(
        body,
        mesh=_MESH,
        out_type=jax.ShapeDtypeStruct((NC, NPAD, H), jnp.float32),
        scratch_types=[
            pltpu.VMEM((ROWS_PER_W, K), jnp.int32),
            pltpu.VMEM((ROWS_PER_W, K), jnp.int32),
            pltpu.VMEM((ROWS_PER_W, K), jnp.float32),
            pltpu.VMEM((NBUF, K, H), jnp.float32),
            pltpu.VMEM_SHARED((NPAD, H), jnp.float32),
            pltpu.SemaphoreType.DMA((NBUF,)),
            pltpu.SemaphoreType.DMA((NBUF,)),
        ],
        compiler_params=pltpu.CompilerParams(needs_layout_passes=False, use_tc_tiling_on_sc=False),
    )


_layer_8 = _make_layer_kernel(8)
_layer_16 = _make_layer_kernel(16)


def _elu(v):
    return jnp.where(v > 0, v, jnp.exp(v) - 1.0)


def _tc1_body(degp_ref, x_ref, w1_ref, dinv_ref, y1_ref):
    deg = degp_ref[0, :] + degp_ref[1, :] + 1.0
    dinv = lax.rsqrt(deg)
    dinv_ref[...] = dinv[:, None]
    y1_ref[...] = jnp.dot(x_ref[...], w1_ref[...],
                          preferred_element_type=jnp.float32) * dinv[:, None]


def _tc_mid_body(sp_ref, y_ref, dinv_ref, b_ref, w_ref, yn_ref):
    dinv = dinv_ref[...]
    h = _elu(dinv * (sp_ref[0] + sp_ref[1] + y_ref[...]) + b_ref[...][None, :])
    yn_ref[...] = jnp.dot(h, w_ref[...],
                          preferred_element_type=jnp.float32) * dinv


def _tc_final_body(sp_ref, y_ref, dinv_ref, b_ref, out_ref):
    out_ref[...] = (dinv_ref[...] * (sp_ref[0] + sp_ref[1] + y_ref[...])
                    + b_ref[...][None, :])


def kernel(x, edge_index, edge_attr, W1, b1, W2, b2, W3, b3):
    padi = jnp.zeros((EPAD - E,), jnp.int32)
    row2d = jnp.concatenate([edge_index[0], padi]).reshape(NW, ROWS_PER_W, K)
    col2d = jnp.concatenate([edge_index[1], padi]).reshape(NW, ROWS_PER_W, K)
    ew2d = jnp.concatenate([edge_attr, jnp.zeros((EPAD - E,), jnp.float32)]
                           ).reshape(NW, ROWS_PER_W, K)
    zero1 = jnp.zeros((NSLC,), jnp.float32)

    deg_parts = _deg_kernel(col2d, ew2d, zero1)
    degp = deg_parts[:, :N]

    dinv, y1 = pl.pallas_call(
        _tc1_body,
        out_shape=(jax.ShapeDtypeStruct((N, 1), jnp.float32),
                   jax.ShapeDtypeStruct((N, 8), jnp.float32)),
    )(degp, x, W1)

    s1 = _layer_8(y1, row2d, col2d, ew2d, jnp.zeros((NSLC, 8), jnp.float32))
    W2p = jnp.pad(W2, ((0, 0), (0, 4)))
    y2p = pl.pallas_call(
        _tc_mid_body,
        out_shape=jax.ShapeDtypeStruct((N, 8), jnp.float32),
    )(s1[:, :N, :], y1, dinv, b1, W2p)

    s2 = _layer_8(y2p, row2d, col2d, ew2d, jnp.zeros((NSLC, 8), jnp.float32))
    y3 = pl.pallas_call(
        _tc_mid_body,
        out_shape=jax.ShapeDtypeStruct((N, 16), jnp.float32),
    )(s2[:, :N, :4], y2p[:, :4], dinv, b2, W3)

    s3 = _layer_16(y3, row2d, col2d, ew2d, jnp.zeros((NSLC, 16), jnp.float32))
    out = pl.pallas_call(
        _tc_final_body,
        out_shape=jax.ShapeDtypeStruct((N, 16), jnp.float32),
    )(s3[:, :N, :], y3, dinv, b3)
    return out


# speedup vs baseline: 29.8432x; 1.0421x over previous
import functools

import jax
import jax.numpy as jnp
from jax import lax
from jax.experimental import pallas as pl
from jax.experimental.pallas import tpu as pltpu
from jax.experimental.pallas import tpu_sc as plsc

N = 10000
E = 320000
D = 128

NC = 2
NS = 16
LANES = 16
NW = NC * NS
K = 128
ROWS_PER_W = 80
CHUNKS = ROWS_PER_W
EPAD = NW * ROWS_PER_W * K
NBUF = 8
OUTER = CHUNKS // NBUF
NPAD = 10240
NSLC = NPAD // NS

_MESH = plsc.VectorSubcoreMesh(core_axis_name="c", subcore_axis_name="s")


def _deg_body(col_hbm, ew_hbm, zero_hbm, out_hbm, colv, ewv, acc):
    cid = lax.axis_index("c")
    sid = lax.axis_index("s")
    wid = sid * NC + cid
    pltpu.sync_copy(zero_hbm, acc.at[pl.ds(sid * NSLC, NSLC)])
    plsc.subcore_barrier()
    pltpu.sync_copy(col_hbm.at[wid], colv)
    pltpu.sync_copy(ew_hbm.at[wid], ewv)

    def chunk(j, carry):
        pltpu.sync_copy(ewv.at[j], acc.at[colv.at[j]], add=True)
        return carry

    lax.fori_loop(0, CHUNKS, chunk, 0)
    plsc.subcore_barrier()
    pltpu.sync_copy(acc.at[pl.ds(sid * NSLC, NSLC)],
                    out_hbm.at[cid, pl.ds(sid * NSLC, NSLC)])


@functools.partial(
    pl.kernel,
    mesh=_MESH,
    out_type=jax.ShapeDtypeStruct((NC, NPAD), jnp.float32),
    scratch_types=[
        pltpu.VMEM((ROWS_PER_W, K), jnp.int32),
        pltpu.VMEM((ROWS_PER_W, K), jnp.float32),
        pltpu.VMEM_SHARED((NPAD,), jnp.float32),
    ],
    compiler_params=pltpu.CompilerParams(needs_layout_passes=False, use_tc_tiling_on_sc=False),
)
def _deg_kernel(col_hbm, ew_hbm, zero_hbm, out_hbm, colv, ewv, acc):
    _deg_body(col_hbm, ew_hbm, zero_hbm, out_hbm, colv, ewv, acc)


def _make_layer_kernel(H):
    groups = (K * H) // LANES

    def body(y_hbm, row_hbm, col_hbm, ew_hbm, zero_hbm, out_hbm,
             rowv, colv, ewv, gbuf, acc, semg, sems):
        cid = lax.axis_index("c")
        sid = lax.axis_index("s")
        wid = sid * NC + cid
        pltpu.sync_copy(zero_hbm, acc.at[pl.ds(sid * NSLC, NSLC)])
        plsc.subcore_barrier()
        pltpu.sync_copy(row_hbm.at[wid], rowv)
        pltpu.sync_copy(col_hbm.at[wid], colv)
        pltpu.sync_copy(ew_hbm.at[wid], ewv)

        lanes = lax.iota(jnp.int32, LANES)
        log2h = H.bit_length() - 1
        e_base = lax.shift_right_logical(lanes, log2h)
        h_vec = lax.bitwise_and(lanes, H - 1)

        def chunk_group(i, carry):
            base = i * NBUF
            gds = [pltpu.async_copy(y_hbm.at[rowv.at[base + b]], gbuf.at[b],
                                    semg.at[b]) for b in range(NBUF)]
            sds = []
            for b in range(NBUF):
                gds[b].wait()
                gb = gbuf.at[b]
                ew_row = ewv.at[base + b]
                if H == LANES:
                    for g in range(groups):
                        if g % LANES == 0:
                            ew16 = ew_row[pl.ds(g, LANES)]
                        gb[g, :] = gb[g, :] * ew16[g % LANES]
                else:
                    for g in range(groups):
                        e_vec = e_base + (LANES * g) // H
                        v = plsc.load_gather(gb, [e_vec, h_vec])
                        s = plsc.load_gather(ew_row, [e_vec])
                        plsc.store_scatter(gb, [e_vec, h_vec], v * s)
                sds.append(pltpu.async_copy(gb, acc.at[colv.at[base + b]],
                                            sems.at[b], add=True))
            for sd in sds:
                sd.wait()
            return carry

        lax.fori_loop(0, OUTER, chunk_group, 0)
        plsc.subcore_barrier()
        pltpu.sync_copy(acc.at[pl.ds(sid * NSLC, NSLC)],
                        out_hbm.at[cid, pl.ds(sid * NSLC, NSLC)])

    return pl.kernel(
        body,
        mesh=_MESH,
        out_type=jax.ShapeDtypeStruct((NC, NPAD, H), jnp.float32),
        scratch_types=[
            pltpu.VMEM((ROWS_PER_W, K), jnp.int32),
            pltpu.VMEM((ROWS_PER_W, K), jnp.int32),
            pltpu.VMEM((ROWS_PER_W, K), jnp.float32),
            pltpu.VMEM((NBUF, K, H), jnp.float32),
            pltpu.VMEM_SHARED((NPAD, H), jnp.float32),
            pltpu.SemaphoreType.DMA((NBUF,)),
            pltpu.SemaphoreType.DMA((NBUF,)),
        ],
        compiler_params=pltpu.CompilerParams(needs_layout_passes=False, use_tc_tiling_on_sc=False),
    )


_layer_8 = _make_layer_kernel(8)
_layer_16 = _make_layer_kernel(16)


def _elu(v):
    return jnp.where(v > 0, v, jnp.exp(v) - 1.0)


def _tc1_body(degp_ref, x_ref, w1_ref, dinv_ref, y1_ref):
    deg = degp_ref[0, :] + degp_ref[1, :] + 1.0
    dinv = lax.rsqrt(deg)
    dinv_ref[...] = dinv[:, None]
    y1_ref[...] = jnp.dot(x_ref[...], w1_ref[...],
                          preferred_element_type=jnp.float32) * dinv[:, None]


def _tc_mid_body(sp_ref, y_ref, dinv_ref, b_ref, w_ref, yn_ref):
    dinv = dinv_ref[...]
    h = _elu(dinv * (sp_ref[0] + sp_ref[1] + y_ref[...]) + b_ref[...][None, :])
    yn_ref[...] = jnp.dot(h, w_ref[...],
                          preferred_element_type=jnp.float32) * dinv


def _tc_final_body(sp_ref, y_ref, dinv_ref, b_ref, out_ref):
    out_ref[...] = (dinv_ref[...] * (sp_ref[0] + sp_ref[1] + y_ref[...])
                    + b_ref[...][None, :])


def kernel(x, edge_index, edge_attr, W1, b1, W2, b2, W3, b3):
    padi = jnp.zeros((EPAD - E,), jnp.int32)
    row2d = jnp.concatenate([edge_index[0], padi]).reshape(NW, ROWS_PER_W, K)
    col2d = jnp.concatenate([edge_index[1], padi]).reshape(NW, ROWS_PER_W, K)
    ew2d = jnp.concatenate([edge_attr, jnp.zeros((EPAD - E,), jnp.float32)]
                           ).reshape(NW, ROWS_PER_W, K)
    zero1 = jnp.zeros((NSLC,), jnp.float32)

    deg_parts = _deg_kernel(col2d, ew2d, zero1)
    degp = deg_parts[:, :N]

    dinv, y1 = pl.pallas_call(
        _tc1_body,
        out_shape=(jax.ShapeDtypeStruct((N, 1), jnp.float32),
                   jax.ShapeDtypeStruct((N, 8), jnp.float32)),
    )(degp, x, W1)

    s1 = _layer_8(y1, row2d, col2d, ew2d, jnp.zeros((NSLC, 8), jnp.float32))
    W2p = jnp.pad(W2, ((0, 0), (0, 4)))
    y2p = pl.pallas_call(
        _tc_mid_body,
        out_shape=jax.ShapeDtypeStruct((N, 8), jnp.float32),
    )(s1[:, :N, :], y1, dinv, b1, W2p)

    s2 = _layer_8(y2p, row2d, col2d, ew2d, jnp.zeros((NSLC, 8), jnp.float32))
    y3 = pl.pallas_call(
        _tc_mid_body,
        out_shape=jax.ShapeDtypeStruct((N, 16), jnp.float32),
    )(s2[:, :N, :4], y2p[:, :4], dinv, b2, W3)

    s3 = _layer_16(y3, row2d, col2d, ew2d, jnp.zeros((NSLC, 16), jnp.float32))
    out = pl.pallas_call(
        _tc_final_body,
        out_shape=jax.ShapeDtypeStruct((N, 16), jnp.float32),
    )(s3[:, :N, :], y3, dinv, b3)
    return out
